# Initial kernel scaffold; baseline (speedup 1.0000x reference)
import jax
import jax.numpy as jnp
from jax.experimental import pallas as pl


def kernel(x, pos, batch, params):
    raise NotImplementedError("write your pallas kernel here")


# speedup vs baseline: 1.3189x; 1.3189x over previous
import functools

import jax
import jax.numpy as jnp
from jax.experimental import pallas as pl
from jax.experimental.pallas import tpu as pltpu

_B = 8
_P = 2048
_K = 64
_R1, _R2 = 0.2, 0.4
_N1, _N2 = 1024, 256
_NEG = -1e30


def _prep_layers(layers):
    out = []
    for (W, b, g, be, m, v) in layers:
        scale = g * jax.lax.rsqrt(v + 1e-5)
        shift = be - m * scale
        out.append((W.T, b[None, :], scale[None, :], shift[None, :]))
    return out


def _fps_body(px_ref, py_ref, pz_ref,
              cx1_ref, cy1_ref, cz1_ref, cx2_ref, cy2_ref, cz2_ref):
    def run_level(xs, ys, zs, n, cxr, cyr, czr):
        Bn, Pn = xs.shape
        iota = jax.lax.broadcasted_iota(jnp.int32, (Bn, Pn), 1)
        iota_n = jax.lax.broadcasted_iota(jnp.int32, (Bn, n), 1)
        cx = xs[:, 0:1]
        cy = ys[:, 0:1]
        cz = zs[:, 0:1]
        hot0 = (iota_n == 0).astype(jnp.float32)
        acc_x = cx * hot0
        acc_y = cy * hot0
        acc_z = cz * hot0
        dists0 = jnp.full((Bn, Pn), jnp.inf, jnp.float32)

        def body(i, st):
            dists, cx, cy, cz, ax, ay, az = st
            d = (xs - cx) ** 2 + (ys - cy) ** 2 + (zs - cz) ** 2
            dists = jnp.minimum(dists, d)
            m = jnp.max(dists, axis=1, keepdims=True)
            nxt = jnp.min(jnp.where(dists == m, iota, Pn), axis=1,
                          keepdims=True)
            sel = (iota == nxt).astype(jnp.float32)
            cx = jnp.sum(xs * sel, axis=1, keepdims=True)
            cy = jnp.sum(ys * sel, axis=1, keepdims=True)
            cz = jnp.sum(zs * sel, axis=1, keepdims=True)
            hot = (iota_n == i).astype(jnp.float32)
            ax = ax + cx * hot
            ay = ay + cy * hot
            az = az + cz * hot
            return (dists, cx, cy, cz, ax, ay, az)

        st = jax.lax.fori_loop(
            1, n, body, (dists0, cx, cy, cz, acc_x, acc_y, acc_z))
        cxr[...] = st[4]
        cyr[...] = st[5]
        czr[...] = st[6]

    run_level(px_ref[...], py_ref[...], pz_ref[...], _N1,
              cx1_ref, cy1_ref, cz1_ref)
    run_level(cx1_ref[...], cy1_ref[...], cz1_ref[...], _N2,
              cx2_ref, cy2_ref, cz2_ref)


def _fps_call(px, py, pz):
    outs = [jax.ShapeDtypeStruct((_B, _N1), jnp.float32)] * 3 + \
           [jax.ShapeDtypeStruct((_B, _N2), jnp.float32)] * 3
    return pl.pallas_call(
        _fps_body,
        out_shape=outs,
    )(px, py, pz)


def _mlp_max_body(nlayers, Kn, block_c, fin, f_ref, m_ref, *refs):
    out_ref = refs[-1]
    prefs = refs[:-1]
    z = f_ref[...].reshape(Kn * block_c, fin)
    for li in range(nlayers):
        wt = prefs[4 * li][...]
        b = prefs[4 * li + 1][...]
        sc = prefs[4 * li + 2][...]
        sh = prefs[4 * li + 3][...]
        z = jnp.dot(z, wt, preferred_element_type=jnp.float32) + b
        z = jnp.maximum(z, 0.0) * sc + sh
    msk = m_ref[...].reshape(Kn * block_c, 1)
    z = jnp.where(msk > 0.0, z, _NEG)
    rows = Kn * block_c
    while rows > block_c:
        rows //= 2
        z = jnp.maximum(z[:rows], z[rows:])
        msk = jnp.maximum(msk[:rows], msk[rows:])
    out_ref[...] = jnp.where(msk > 0.0, z, 0.0)


def _mlp_max_call(feat_t, mask_t, players, block_c):
    Kn, N, fin = feat_t.shape
    fout = players[-1][0].shape[1]
    nblk = N // block_c
    body = functools.partial(_mlp_max_body, len(players), Kn, block_c, fin)
    in_specs = [
        pl.BlockSpec((Kn, block_c, fin), lambda i: (0, i, 0)),
        pl.BlockSpec((Kn, block_c, 1), lambda i: (0, i, 0)),
    ]
    args = [feat_t, mask_t]
    for p in players:
        for a in p:
            in_specs.append(pl.BlockSpec(a.shape, lambda i: (0, 0)))
            args.append(a)
    return pl.pallas_call(
        body,
        grid=(nblk,),
        in_specs=in_specs,
        out_specs=pl.BlockSpec((block_c, fout), lambda i: (i, 0)),
        out_shape=jax.ShapeDtypeStruct((N, fout), jnp.float32),
    )(*args)


def _global_body(f_ref, *refs):
    out_ref = refs[-1]
    prefs = refs[:-1]
    z = f_ref[...]
    for li in range(3):
        wt = prefs[4 * li][...]
        b = prefs[4 * li + 1][...]
        sc = prefs[4 * li + 2][...]
        sh = prefs[4 * li + 3][...]
        z = jnp.dot(z, wt, preferred_element_type=jnp.float32) + b
        z = jnp.maximum(z, 0.0) * sc + sh
    rows = z.shape[0]
    while rows > _B:
        rows //= 2
        z = jnp.maximum(z[:rows], z[rows:])
    base = 12
    for li in range(3):
        wt = prefs[base + 2 * li][...]
        b = prefs[base + 2 * li + 1][...]
        z = jnp.dot(z, wt, preferred_element_type=jnp.float32) + b
        if li < 2:
            z = jnp.maximum(z, 0.0)
    out_ref[...] = z


def _global_call(feat_t, players3, lins):
    args = [feat_t]
    for p in players3:
        args.extend(p)
    for (W, b) in lins:
        args.append(W.T)
        args.append(b[None, :])
    return pl.pallas_call(
        _global_body,
        out_shape=jax.ShapeDtypeStruct((_B, 32), jnp.float32),
    )(*args)


def _radius_nbrs(pos, centers, r, k):
    d2 = jnp.sum((centers[:, None, :] - pos[None, :, :]) ** 2, axis=-1)
    Pn = pos.shape[0]
    mask = d2 <= r * r
    keyv = jnp.where(mask, jnp.arange(Pn)[None, :], Pn)
    negv, idx = jax.lax.top_k(-keyv, k)
    valid = (-negv) < Pn
    idx = jnp.where(valid, idx, 0)
    return idx, valid


def _sa_stage(xb, posb, centers, r, players, fin):
    def one(xx, pp, cc):
        idx, valid = _radius_nbrs(pp, cc, r, _K)
        xj = xx[idx]
        rel = pp[idx] - cc[:, None, :]
        feat = jnp.concatenate([xj, rel], axis=-1)
        return feat, valid

    feat, valid = jax.vmap(one)(xb, posb, centers)
    n = centers.shape[1]
    feat_t = feat.transpose(2, 0, 1, 3).reshape(_K, _B * n, fin)
    mask_t = valid.transpose(2, 0, 1).reshape(_K, _B * n, 1)
    mask_t = mask_t.astype(jnp.float32)
    block_c = 128 if n >= 128 else n
    out = _mlp_max_call(feat_t, mask_t, players, block_c)
    return out.reshape(_B, n, -1)


def kernel(x, pos, batch, params):
    p1l = _prep_layers(params['mlp1'])
    p2l = _prep_layers(params['mlp2'])
    p3l = _prep_layers(params['mlp3'])

    xb = x.reshape(_B, _P, 3)
    pb = pos.reshape(_B, _P, 3)
    px = pb[:, :, 0]
    py = pb[:, :, 1]
    pz = pb[:, :, 2]
    cx1, cy1, cz1, cx2, cy2, cz2 = _fps_call(px, py, pz)
    centers1 = jnp.stack([cx1, cy1, cz1], axis=-1)
    centers2 = jnp.stack([cx2, cy2, cz2], axis=-1)

    x1 = _sa_stage(xb, pb, centers1, _R1, p1l, 6)
    x2 = _sa_stage(x1, centers1, centers2, _R2, p2l, 131)

    feat3 = jnp.concatenate([x2, centers2], axis=-1)
    feat3_t = feat3.transpose(1, 0, 2).reshape(_N2 * _B, 259)
    lins = [params['lin1'], params['lin2'], params['lin3']]
    return _global_call(feat3_t, p3l, lins)


# speedup vs baseline: 23.9465x; 18.1566x over previous
import functools

import jax
import jax.numpy as jnp
from jax.experimental import pallas as pl
from jax.experimental.pallas import tpu as pltpu
from jax.experimental.pallas import tpu_sc as plsc

_B = 8
_P = 2048
_K = 64
_R1, _R2 = 0.2, 0.4
_N1, _N2 = 1024, 256
_NEG = -1e30
_NW = 32


def _prep_layers(layers):
    out = []
    for (W, b, g, be, m, v) in layers:
        scale = g * jax.lax.rsqrt(v + 1e-5)
        shift = be - m * scale
        out.append((W.T, b[None, :], scale[None, :], shift[None, :]))
    return out


def _fps_body(px_ref, py_ref, pz_ref,
              cx1_ref, cy1_ref, cz1_ref, cx2_ref, cy2_ref, cz2_ref):
    def run_level(xs, ys, zs, n, cxr, cyr, czr):
        Bn, Pn = xs.shape
        iota = jax.lax.broadcasted_iota(jnp.int32, (Bn, Pn), 1)
        iota_n = jax.lax.broadcasted_iota(jnp.int32, (Bn, n), 1)
        cx = xs[:, 0:1]
        cy = ys[:, 0:1]
        cz = zs[:, 0:1]
        hot0 = (iota_n == 0).astype(jnp.float32)
        acc_x = cx * hot0
        acc_y = cy * hot0
        acc_z = cz * hot0
        dists0 = jnp.full((Bn, Pn), jnp.inf, jnp.float32)

        def body(i, st):
            dists, cx, cy, cz, ax, ay, az = st
            d = (xs - cx) ** 2 + (ys - cy) ** 2 + (zs - cz) ** 2
            dists = jnp.minimum(dists, d)
            m = jnp.max(dists, axis=1, keepdims=True)
            nxt = jnp.min(jnp.where(dists == m, iota, Pn), axis=1,
                          keepdims=True)
            sel = (iota == nxt).astype(jnp.float32)
            cx = jnp.sum(xs * sel, axis=1, keepdims=True)
            cy = jnp.sum(ys * sel, axis=1, keepdims=True)
            cz = jnp.sum(zs * sel, axis=1, keepdims=True)
            hot = (iota_n == i).astype(jnp.float32)
            ax = ax + cx * hot
            ay = ay + cy * hot
            az = az + cz * hot
            return (dists, cx, cy, cz, ax, ay, az)

        st = jax.lax.fori_loop(
            1, n, body, (dists0, cx, cy, cz, acc_x, acc_y, acc_z))
        cxr[...] = st[4]
        cyr[...] = st[5]
        czr[...] = st[6]

    run_level(px_ref[...], py_ref[...], pz_ref[...], _N1,
              cx1_ref, cy1_ref, cz1_ref)
    run_level(cx1_ref[...], cy1_ref[...], cz1_ref[...], _N2,
              cx2_ref, cy2_ref, cz2_ref)


def _fps_call(px, py, pz):
    outs = [jax.ShapeDtypeStruct((_B, _N1), jnp.float32)] * 3 + \
           [jax.ShapeDtypeStruct((_B, _N2), jnp.float32)] * 3
    return pl.pallas_call(
        _fps_body,
        out_shape=outs,
    )(px, py, pz)


def _rank_body(r2, Pn, bc,
               cxt_ref, cyt_ref, czt_ref, px_ref, py_ref, pz_ref, u_ref,
               rank_ref, nv_ref):
    b = pl.program_id(0)
    chot = (jax.lax.broadcasted_iota(jnp.int32, (bc, _B), 1) == b)
    chot = chot.astype(jnp.float32)
    phot = (jax.lax.broadcasted_iota(jnp.int32, (_B, Pn), 0) == b)
    phot = phot.astype(jnp.float32)
    cx = jnp.sum(cxt_ref[...] * chot, axis=1, keepdims=True)
    cy = jnp.sum(cyt_ref[...] * chot, axis=1, keepdims=True)
    cz = jnp.sum(czt_ref[...] * chot, axis=1, keepdims=True)
    px = jnp.sum(px_ref[...] * phot, axis=0, keepdims=True)
    py = jnp.sum(py_ref[...] * phot, axis=0, keepdims=True)
    pz = jnp.sum(pz_ref[...] * phot, axis=0, keepdims=True)
    d2 = (cx - px) ** 2 + (cy - py) ** 2 + (cz - pz) ** 2
    mask = (d2 <= r2).astype(jnp.bfloat16)
    rank = jnp.dot(mask, u_ref[...], preferred_element_type=jnp.float32)
    ranki = rank.astype(jnp.int32)
    rank_ref[...] = ranki.reshape(1, bc, Pn)
    nv_ref[...] = jnp.minimum(ranki[:, Pn - 1:Pn], _K).reshape(1, bc, 1)


def _rank_call(cxt, cyt, czt, px, py, pz, r, bc):
    n, _ = cxt.shape
    Pn = px.shape[1]
    u = jnp.triu(jnp.ones((Pn, Pn), jnp.bfloat16))
    body = functools.partial(_rank_body, r * r, Pn, bc)
    grid = (_B, n // bc)
    return pl.pallas_call(
        body,
        grid=grid,
        in_specs=[
            pl.BlockSpec((bc, _B), lambda b, i: (i, 0)),
            pl.BlockSpec((bc, _B), lambda b, i: (i, 0)),
            pl.BlockSpec((bc, _B), lambda b, i: (i, 0)),
            pl.BlockSpec((_B, Pn), lambda b, i: (0, 0)),
            pl.BlockSpec((_B, Pn), lambda b, i: (0, 0)),
            pl.BlockSpec((_B, Pn), lambda b, i: (0, 0)),
            pl.BlockSpec((Pn, Pn), lambda b, i: (0, 0)),
        ],
        out_specs=[
            pl.BlockSpec((1, bc, Pn), lambda b, i: (b, i, 0)),
            pl.BlockSpec((1, bc, 1), lambda b, i: (b, i, 0)),
        ],
        out_shape=[
            jax.ShapeDtypeStruct((_B, n, Pn), jnp.int32),
            jax.ShapeDtypeStruct((_B, n, 1), jnp.int32),
        ],
    )(cxt, cyt, czt, px, py, pz, u)


_SC_MESH = plsc.VectorSubcoreMesh(core_axis_name="c", subcore_axis_name="s",
                                  num_cores=2, num_subcores=16)


def _bsearch(buf, row_splat, sv, Pn):
    c = jnp.zeros((16,), jnp.int32)
    w = Pn
    while w >= 1:
        pi = c + (w - 1)
        ok = pi < Pn
        pic = jnp.minimum(pi, Pn - 1)
        rv = plsc.load_gather(buf, [row_splat, pic])
        ok = ok & (rv <= sv)
        c = c + jnp.where(ok, w, 0)
        w //= 2
    return c


def _sc_sa1(rank2d, t1f, c3f):
    N, Pn = _B * _N1, _P
    cpt = N // _NW
    RB = 4
    nblocks = cpt // RB
    n_per_cloud = _N1

    def body(rank_hbm, t1_hbm, c3_hbm, out_hbm,
             tvm, cvm, buf_a, buf_b, rowbuf, sem_a, sem_b, sem_t, sem_o):
        wid = jax.lax.axis_index("s") * 2 + jax.lax.axis_index("c")
        base = wid * cpt
        cloud = base // n_per_cloud
        lane = jax.lax.iota(jnp.int32, 16)

        pltpu.async_copy(t1_hbm.at[pl.ds(cloud * Pn * 8, Pn * 8)], tvm,
                         sem_t).wait()
        pltpu.async_copy(c3_hbm.at[pl.ds(base * 8, cpt * 8)], cvm,
                         sem_t).wait()

        def start(blk, buf, sem):
            blk = jnp.minimum(blk, nblocks - 1)
            pltpu.async_copy(rank_hbm.at[pl.ds(base + blk * RB, RB)], buf,
                             sem)

        def wait(buf, sem):
            pltpu.make_async_copy(rank_hbm.at[pl.ds(0, RB)], buf, sem).wait()

        def process(blk, buf):
            for cc in range(RB):
                g = base + blk * RB + cc
                local = g - base
                lastv = buf[cc, pl.ds(Pn - 16, 16)]
                nval = jnp.minimum(jnp.max(lastv), _K)
                row_splat = jnp.full((16,), cc, jnp.int32)
                lsplat = jnp.full((16,), local, jnp.int32)
                ccx = plsc.load_gather(cvm, [lsplat * 8])
                ccy = plsc.load_gather(cvm, [lsplat * 8 + 1])
                ccz = plsc.load_gather(cvm, [lsplat * 8 + 2])
                for q in range(4):
                    sv = lane + (16 * q)
                    c = _bsearch(buf, row_splat, sv, Pn)
                    nbr = jnp.where(sv < nval, c, 0) * 8
                    xj0 = plsc.load_gather(tvm, [nbr])
                    xj1 = plsc.load_gather(tvm, [nbr + 1])
                    xj2 = plsc.load_gather(tvm, [nbr + 2])
                    pj0 = plsc.load_gather(tvm, [nbr + 3])
                    pj1 = plsc.load_gather(tvm, [nbr + 4])
                    pj2 = plsc.load_gather(tvm, [nbr + 5])
                    rows = (lane + (cc * _K + q * 16)) * 8
                    plsc.store_scatter(rowbuf, [rows], xj0)
                    plsc.store_scatter(rowbuf, [rows + 1], xj1)
                    plsc.store_scatter(rowbuf, [rows + 2], xj2)
                    plsc.store_scatter(rowbuf, [rows + 3], pj0 - ccx)
                    plsc.store_scatter(rowbuf, [rows + 4], pj1 - ccy)
                    plsc.store_scatter(rowbuf, [rows + 5], pj2 - ccz)
            g0 = base + blk * RB
            pltpu.async_copy(rowbuf,
                             out_hbm.at[pl.ds(g0 * _K * 8, RB * _K * 8)],
                             sem_o).wait()

        z16 = jnp.zeros((16,), jnp.float32)
        for r in range(RB * _K // 16):
            rows0 = (lane + r * 16) * 8
            plsc.store_scatter(rowbuf, [rows0 + 6], z16)
            plsc.store_scatter(rowbuf, [rows0 + 7], z16)

        start(0, buf_a, sem_a)
        start(1, buf_b, sem_b)

        def loop_body(i, carry):
            wait(buf_a, sem_a)
            process(2 * i, buf_a)
            start(2 * i + 2, buf_a, sem_a)
            wait(buf_b, sem_b)
            process(2 * i + 1, buf_b)
            start(2 * i + 3, buf_b, sem_b)
            return carry

        jax.lax.fori_loop(0, nblocks // 2, loop_body, 0)
        wait(buf_a, sem_a)
        wait(buf_b, sem_b)

    kern = pl.kernel(
        body,
        out_type=pltpu.HBM((_K * N * 8,), jnp.float32),
        mesh=_SC_MESH,
        compiler_params=pltpu.CompilerParams(needs_layout_passes=False),
        scratch_types=[
            pltpu.VMEM((_P * 8,), jnp.float32),
            pltpu.VMEM((cpt * 8,), jnp.float32),
            pltpu.VMEM((RB, Pn), jnp.int32),
            pltpu.VMEM((RB, Pn), jnp.int32),
            pltpu.VMEM((RB * _K * 8,), jnp.float32),
            pltpu.SemaphoreType.DMA,
            pltpu.SemaphoreType.DMA,
            pltpu.SemaphoreType.DMA,
            pltpu.SemaphoreType.DMA,
        ],
    )
    return kern(rank2d, t1f, c3f)


def _sc_sa2(rank2d, xt, p1f, c3f):
    N, Pn = _B * _N2, _N1
    cpt = N // _NW
    RB = 4
    nblocks = cpt // RB
    n_per_cloud = _N2

    def body(rank_hbm, xt_hbm, p1_hbm, c3_hbm, outx_hbm, outr_hbm,
             pvm, cvm, buf_a, buf_b, nbrbuf, rowsbuf, relbuf,
             sem_a, sem_b, sem_t, sem_g, sem_o):
        wid = jax.lax.axis_index("s") * 2 + jax.lax.axis_index("c")
        base = wid * cpt
        cloud = base // n_per_cloud
        lane = jax.lax.iota(jnp.int32, 16)

        pltpu.async_copy(p1_hbm.at[pl.ds(cloud * Pn * 4, Pn * 4)], pvm,
                         sem_t).wait()
        pltpu.async_copy(c3_hbm.at[pl.ds(base * 8, cpt * 8)], cvm,
                         sem_t).wait()

        z16 = jnp.zeros((16,), jnp.float32)
        def zinit(r, carry):
            plsc.store_scatter(relbuf, [lane + r * 16], z16)
            return carry
        jax.lax.fori_loop(0, RB * _K, zinit, 0)

        def start(blk, buf, sem):
            blk = jnp.minimum(blk, nblocks - 1)
            pltpu.async_copy(rank_hbm.at[pl.ds(base + blk * RB, RB)], buf,
                             sem)

        def wait(buf, sem):
            pltpu.make_async_copy(rank_hbm.at[pl.ds(0, RB)], buf, sem).wait()

        def process(blk, buf):
            for cc in range(RB):
                g = base + blk * RB + cc
                local = g - base
                lastv = buf[cc, pl.ds(Pn - 16, 16)]
                nval = jnp.minimum(jnp.max(lastv), _K)
                row_splat = jnp.full((16,), cc, jnp.int32)
                lsplat = jnp.full((16,), local, jnp.int32)
                ccx = plsc.load_gather(cvm, [lsplat * 8])
                ccy = plsc.load_gather(cvm, [lsplat * 8 + 1])
                ccz = plsc.load_gather(cvm, [lsplat * 8 + 2])
                for q in range(4):
                    sv = lane + (16 * q)
                    c = _bsearch(buf, row_splat, sv, Pn)
                    nbr = jnp.where(sv < nval, c, 0)
                    pj0 = plsc.load_gather(pvm, [nbr * 4])
                    pj1 = plsc.load_gather(pvm, [nbr * 4 + 1])
                    pj2 = plsc.load_gather(pvm, [nbr * 4 + 2])
                    rows = (lane + (cc * _K + q * 16)) * 16
                    plsc.store_scatter(relbuf, [rows], pj0 - ccx)
                    plsc.store_scatter(relbuf, [rows + 1], pj1 - ccy)
                    plsc.store_scatter(relbuf, [rows + 2], pj2 - ccz)
                    pos = cc * _K + q * 16
                    nbrbuf[pos // 128, pl.ds(pos % 128, 16)] = nbr + cloud * Pn
            g0 = base + blk * RB
            for h in range(RB * _K // 128):
                pltpu.async_copy(xt_hbm.at[nbrbuf.at[h]], rowsbuf,
                                 sem_g).wait()
                pltpu.async_copy(rowsbuf,
                                 outx_hbm.at[pl.ds(g0 * _K + h * 128, 128)],
                                 sem_o).wait()
            pltpu.async_copy(relbuf,
                             outr_hbm.at[pl.ds(g0 * _K * 16, RB * _K * 16)],
                             sem_o).wait()

        start(0, buf_a, sem_a)
        start(1, buf_b, sem_b)

        def loop_body(i, carry):
            wait(buf_a, sem_a)
            process(2 * i, buf_a)
            start(2 * i + 2, buf_a, sem_a)
            wait(buf_b, sem_b)
            process(2 * i + 1, buf_b)
            start(2 * i + 3, buf_b, sem_b)
            return carry

        jax.lax.fori_loop(0, nblocks // 2, loop_body, 0)
        wait(buf_a, sem_a)
        wait(buf_b, sem_b)

    kern = pl.kernel(
        body,
        out_type=[
            pltpu.HBM((_K * N, 128), jnp.float32),
            pltpu.HBM((_K * N * 16,), jnp.float32),
        ],
        mesh=_SC_MESH,
        compiler_params=pltpu.CompilerParams(needs_layout_passes=False),
        scratch_types=[
            pltpu.VMEM((_N1 * 4,), jnp.float32),
            pltpu.VMEM((cpt * 8,), jnp.float32),
            pltpu.VMEM((RB, Pn), jnp.int32),
            pltpu.VMEM((RB, Pn), jnp.int32),
            pltpu.VMEM((RB * _K // 128, 128), jnp.int32),
            pltpu.VMEM((128, 128), jnp.float32),
            pltpu.VMEM((RB * _K * 16,), jnp.float32),
            pltpu.SemaphoreType.DMA,
            pltpu.SemaphoreType.DMA,
            pltpu.SemaphoreType.DMA,
            pltpu.SemaphoreType.DMA,
            pltpu.SemaphoreType.DMA,
        ],
    )
    return kern(rank2d, xt, p1f, c3f)


def _mlp_max_body(nlayers, Kn, bc, fin, f_ref, nv_ref, *refs):
    out_ref = refs[-1]
    prefs = refs[:-1]
    z = f_ref[...].reshape(bc * Kn, fin)
    for li in range(nlayers):
        wt = prefs[4 * li][...]
        b = prefs[4 * li + 1][...]
        sc = prefs[4 * li + 2][...]
        sh = prefs[4 * li + 3][...]
        z = jnp.dot(z, wt, preferred_element_type=jnp.float32) + b
        z = jnp.maximum(z, 0.0) * sc + sh
    fout = z.shape[1]
    z3 = z.reshape(bc, Kn, fout)
    nv = nv_ref[...]
    slot3 = jax.lax.broadcasted_iota(jnp.int32, (bc, Kn, 1), 1)
    nv3 = jnp.broadcast_to(nv[:, :, None], (bc, 1, 1))
    z3 = jnp.where(slot3 < nv3, z3, _NEG)
    k = Kn
    while k > 1:
        k //= 2
        z3 = jnp.maximum(z3[:, :k], z3[:, k:])
    z2 = z3.reshape(bc, fout)
    out_ref[...] = jnp.where(nv > 0, z2, 0.0)


def _mlp_max_call(feats, nvalid, players, bc):
    N, Kn, fin = feats.shape
    fout = players[-1][0].shape[1]
    nblk = N // bc
    body = functools.partial(_mlp_max_body, len(players), Kn, bc, fin)
    in_specs = [
        pl.BlockSpec((bc, Kn, fin), lambda i: (i, 0, 0)),
        pl.BlockSpec((bc, 1), lambda i: (i, 0)),
    ]
    args = [feats, nvalid]
    for p in players:
        for a in p:
            in_specs.append(pl.BlockSpec(a.shape, lambda i: (0, 0)))
            args.append(a)
    return pl.pallas_call(
        body,
        grid=(nblk,),
        in_specs=in_specs,
        out_specs=pl.BlockSpec((bc, fout), lambda i: (i, 0)),
        out_shape=jax.ShapeDtypeStruct((N, fout), jnp.float32),
    )(*args)


def _global_body(f_ref, *refs):
    out_ref = refs[-1]
    prefs = refs[:-1]
    z = f_ref[...]
    for li in range(3):
        wt = prefs[4 * li][...]
        b = prefs[4 * li + 1][...]
        sc = prefs[4 * li + 2][...]
        sh = prefs[4 * li + 3][...]
        z = jnp.dot(z, wt, preferred_element_type=jnp.float32) + b
        z = jnp.maximum(z, 0.0) * sc + sh
    rows = z.shape[0]
    while rows > _B:
        rows //= 2
        z = jnp.maximum(z[:rows], z[rows:])
    base = 12
    for li in range(3):
        wt = prefs[base + 2 * li][...]
        b = prefs[base + 2 * li + 1][...]
        z = jnp.dot(z, wt, preferred_element_type=jnp.float32) + b
        if li < 2:
            z = jnp.maximum(z, 0.0)
    out_ref[...] = z


def _global_call(feat_t, players3, lins):
    args = [feat_t]
    for p in players3:
        args.extend(p)
    for (W, b) in lins:
        args.append(W.T)
        args.append(b[None, :])
    return pl.pallas_call(
        _global_body,
        out_shape=jax.ShapeDtypeStruct((_B, 32), jnp.float32),
    )(*args)


def _pad_first_layer(players, fin, dpad):
    w1t, b1, sc1, sh1 = players[0]
    f1 = w1t.shape[1]
    wpad = jnp.zeros((dpad, f1), jnp.float32)
    wpad = wpad.at[:fin].set(w1t)
    return [(wpad, b1, sc1, sh1)] + list(players[1:])


def kernel(x, pos, batch, params):
    p1l = _prep_layers(params['mlp1'])
    p2l = _prep_layers(params['mlp2'])
    p3l = _prep_layers(params['mlp3'])

    xb = x.reshape(_B, _P, 3)
    pb = pos.reshape(_B, _P, 3)
    px = pb[:, :, 0]
    py = pb[:, :, 1]
    pz = pb[:, :, 2]
    cx1, cy1, cz1, cx2, cy2, cz2 = _fps_call(px, py, pz)

    rank1, nv1 = _rank_call(cx1.T, cy1.T, cz1.T, px, py, pz, _R1, 128)
    t1 = jnp.concatenate([xb, pb, jnp.zeros((_B, _P, 2), jnp.float32)],
                         axis=-1).reshape(_B * _P, 8)
    c1s = jnp.stack([cx1, cy1, cz1], axis=-1).reshape(_B * _N1, 3)
    c13 = jnp.concatenate([c1s, jnp.zeros((_B * _N1, 5), jnp.float32)],
                          axis=1)
    feats1 = _sc_sa1(rank1.reshape(_B * _N1, _P), t1.reshape(-1),
                     c13.reshape(-1))
    feats1 = feats1.reshape(_B * _N1, _K, 8)
    x1 = _mlp_max_call(feats1, nv1.reshape(_B * _N1, 1),
                       _pad_first_layer(p1l, 6, 8), 128)

    rank2, nv2 = _rank_call(cx2.T, cy2.T, cz2.T, cx1, cy1, cz1, _R2, 128)
    p1c = jnp.stack([cx1, cy1, cz1,
                     jnp.zeros((_B, _N1), jnp.float32)], axis=-1)
    c2s = jnp.stack([cx2, cy2, cz2], axis=-1).reshape(_B * _N2, 3)
    c23 = jnp.concatenate([c2s, jnp.zeros((_B * _N2, 5), jnp.float32)],
                          axis=1)
    fx2, fr2 = _sc_sa2(rank2.reshape(_B * _N2, _N1), x1, p1c.reshape(-1),
                       c23.reshape(-1))
    fr2 = fr2.reshape(_K * _B * _N2, 16)
    feats2 = jnp.concatenate([fx2, fr2], axis=-1)
    feats2 = feats2.reshape(_B * _N2, _K, 144)
    pl2 = _pad_first_layer(p2l, 131, 144)
    x2 = _mlp_max_call(feats2, nv2.reshape(_B * _N2, 1), pl2, 128)

    feat3 = jnp.concatenate(
        [x2.reshape(_B, _N2, 256), jnp.stack([cx2, cy2, cz2], axis=-1)],
        axis=-1)
    feat3_t = feat3.transpose(1, 0, 2).reshape(_N2 * _B, 259)
    lins = [params['lin1'], params['lin2'], params['lin3']]
    return _global_call(feat3_t, p3l, lins)


# speedup vs baseline: 24.0759x; 1.0054x over previous
import functools

import jax
import jax.numpy as jnp
from jax.experimental import pallas as pl
from jax.experimental.pallas import tpu as pltpu
from jax.experimental.pallas import tpu_sc as plsc

_B = 8
_P = 2048
_K = 64
_R1, _R2 = 0.2, 0.4
_N1, _N2 = 1024, 256
_NEG = -1e30
_NW = 32


def _prep_layers(layers):
    out = []
    for (W, b, g, be, m, v) in layers:
        scale = g * jax.lax.rsqrt(v + 1e-5)
        shift = be - m * scale
        out.append((W.T, b[None, :], scale[None, :], shift[None, :]))
    return out


def _fps_body(px_ref, py_ref, pz_ref,
              cx1_ref, cy1_ref, cz1_ref, cx2_ref, cy2_ref, cz2_ref):
    def run_level(xs, ys, zs, n, cxr, cyr, czr):
        Bn, Pn = xs.shape
        iota = jax.lax.broadcasted_iota(jnp.int32, (Bn, Pn), 1)
        iota128 = jax.lax.broadcasted_iota(jnp.int32, (Bn, 128), 1)
        cx = xs[:, 0:1]
        cy = ys[:, 0:1]
        cz = zs[:, 0:1]
        dists0 = jnp.full((Bn, Pn), jnp.inf, jnp.float32)

        def outer(bk, st):
            dists, cx, cy, cz = st
            zacc = jnp.zeros((Bn, 128), jnp.float32)

            def inner(j, st2):
                dists, cx, cy, cz, ax, ay, az = st2
                hot = (iota128 == j).astype(jnp.float32)
                ax = ax + cx * hot
                ay = ay + cy * hot
                az = az + cz * hot
                d = (xs - cx) ** 2 + (ys - cy) ** 2 + (zs - cz) ** 2
                dists = jnp.minimum(dists, d)
                m = jnp.max(dists, axis=1, keepdims=True)
                nxt = jnp.min(jnp.where(dists == m, iota, Pn), axis=1,
                              keepdims=True)
                sel = (iota == nxt).astype(jnp.float32)
                cx = jnp.sum(xs * sel, axis=1, keepdims=True)
                cy = jnp.sum(ys * sel, axis=1, keepdims=True)
                cz = jnp.sum(zs * sel, axis=1, keepdims=True)
                return (dists, cx, cy, cz, ax, ay, az)

            dists, cx, cy, cz, ax, ay, az = jax.lax.fori_loop(
                0, 128, inner, (dists, cx, cy, cz, zacc, zacc, zacc))
            base = pl.multiple_of(bk * 128, 128)
            cxr[:, pl.ds(base, 128)] = ax
            cyr[:, pl.ds(base, 128)] = ay
            czr[:, pl.ds(base, 128)] = az
            return (dists, cx, cy, cz)

        jax.lax.fori_loop(0, n // 128, outer, (dists0, cx, cy, cz))

    run_level(px_ref[...], py_ref[...], pz_ref[...], _N1,
              cx1_ref, cy1_ref, cz1_ref)
    run_level(cx1_ref[...], cy1_ref[...], cz1_ref[...], _N2,
              cx2_ref, cy2_ref, cz2_ref)


def _fps_call(px, py, pz):
    outs = [jax.ShapeDtypeStruct((_B, _N1), jnp.float32)] * 3 + \
           [jax.ShapeDtypeStruct((_B, _N2), jnp.float32)] * 3
    return pl.pallas_call(
        _fps_body,
        out_shape=outs,
    )(px, py, pz)


def _rank_body(r2, Pn, bc,
               cxt_ref, cyt_ref, czt_ref, px_ref, py_ref, pz_ref, u_ref,
               rank_ref, nv_ref):
    b = pl.program_id(0)
    chot = (jax.lax.broadcasted_iota(jnp.int32, (bc, _B), 1) == b)
    chot = chot.astype(jnp.float32)
    phot = (jax.lax.broadcasted_iota(jnp.int32, (_B, Pn), 0) == b)
    phot = phot.astype(jnp.float32)
    cx = jnp.sum(cxt_ref[...] * chot, axis=1, keepdims=True)
    cy = jnp.sum(cyt_ref[...] * chot, axis=1, keepdims=True)
    cz = jnp.sum(czt_ref[...] * chot, axis=1, keepdims=True)
    px = jnp.sum(px_ref[...] * phot, axis=0, keepdims=True)
    py = jnp.sum(py_ref[...] * phot, axis=0, keepdims=True)
    pz = jnp.sum(pz_ref[...] * phot, axis=0, keepdims=True)
    d2 = (cx - px) ** 2 + (cy - py) ** 2 + (cz - pz) ** 2
    mask = (d2 <= r2).astype(jnp.bfloat16)
    rank = jnp.dot(mask, u_ref[...], preferred_element_type=jnp.float32)
    ranki = rank.astype(jnp.int32)
    rank_ref[...] = ranki.reshape(1, bc, Pn)
    nv_ref[...] = jnp.minimum(ranki[:, Pn - 1:Pn], _K).reshape(1, bc, 1)


def _rank_call(cxt, cyt, czt, px, py, pz, r, bc):
    n, _ = cxt.shape
    Pn = px.shape[1]
    u = jnp.triu(jnp.ones((Pn, Pn), jnp.bfloat16))
    body = functools.partial(_rank_body, r * r, Pn, bc)
    grid = (_B, n // bc)
    return pl.pallas_call(
        body,
        grid=grid,
        in_specs=[
            pl.BlockSpec((bc, _B), lambda b, i: (i, 0)),
            pl.BlockSpec((bc, _B), lambda b, i: (i, 0)),
            pl.BlockSpec((bc, _B), lambda b, i: (i, 0)),
            pl.BlockSpec((_B, Pn), lambda b, i: (0, 0)),
            pl.BlockSpec((_B, Pn), lambda b, i: (0, 0)),
            pl.BlockSpec((_B, Pn), lambda b, i: (0, 0)),
            pl.BlockSpec((Pn, Pn), lambda b, i: (0, 0)),
        ],
        out_specs=[
            pl.BlockSpec((1, bc, Pn), lambda b, i: (b, i, 0)),
            pl.BlockSpec((1, bc, 1), lambda b, i: (b, i, 0)),
        ],
        out_shape=[
            jax.ShapeDtypeStruct((_B, n, Pn), jnp.int32),
            jax.ShapeDtypeStruct((_B, n, 1), jnp.int32),
        ],
    )(cxt, cyt, czt, px, py, pz, u)


_SC_MESH = plsc.VectorSubcoreMesh(core_axis_name="c", subcore_axis_name="s",
                                  num_cores=2, num_subcores=16)


def _bsearch(buf, row_splat, sv, Pn):
    c = jnp.zeros((16,), jnp.int32)
    w = Pn
    while w >= 1:
        pi = c + (w - 1)
        ok = pi < Pn
        pic = jnp.minimum(pi, Pn - 1)
        rv = plsc.load_gather(buf, [row_splat, pic])
        ok = ok & (rv <= sv)
        c = c + jnp.where(ok, w, 0)
        w //= 2
    return c


def _sc_sa1(rank2d, t1f, c3f):
    N, Pn = _B * _N1, _P
    cpt = N // _NW
    RB = 4
    nblocks = cpt // RB
    n_per_cloud = _N1

    def body(rank_hbm, t1_hbm, c3_hbm, out_hbm,
             tvm, cvm, buf_a, buf_b, rowbuf, sem_a, sem_b, sem_t, sem_o):
        wid = jax.lax.axis_index("s") * 2 + jax.lax.axis_index("c")
        base = wid * cpt
        cloud = base // n_per_cloud
        lane = jax.lax.iota(jnp.int32, 16)

        pltpu.async_copy(t1_hbm.at[pl.ds(cloud * Pn * 8, Pn * 8)], tvm,
                         sem_t).wait()
        pltpu.async_copy(c3_hbm.at[pl.ds(base * 8, cpt * 8)], cvm,
                         sem_t).wait()

        def start(blk, buf, sem):
            blk = jnp.minimum(blk, nblocks - 1)
            pltpu.async_copy(rank_hbm.at[pl.ds(base + blk * RB, RB)], buf,
                             sem)

        def wait(buf, sem):
            pltpu.make_async_copy(rank_hbm.at[pl.ds(0, RB)], buf, sem).wait()

        def process(blk, buf):
            for cc in range(RB):
                g = base + blk * RB + cc
                local = g - base
                lastv = buf[cc, pl.ds(Pn - 16, 16)]
                nval = jnp.minimum(jnp.max(lastv), _K)
                row_splat = jnp.full((16,), cc, jnp.int32)
                lsplat = jnp.full((16,), local, jnp.int32)
                ccx = plsc.load_gather(cvm, [lsplat * 8])
                ccy = plsc.load_gather(cvm, [lsplat * 8 + 1])
                ccz = plsc.load_gather(cvm, [lsplat * 8 + 2])
                for q in range(4):
                    sv = lane + (16 * q)
                    c = _bsearch(buf, row_splat, sv, Pn)
                    nbr = jnp.where(sv < nval, c, 0) * 8
                    xj0 = plsc.load_gather(tvm, [nbr])
                    xj1 = plsc.load_gather(tvm, [nbr + 1])
                    xj2 = plsc.load_gather(tvm, [nbr + 2])
                    pj0 = plsc.load_gather(tvm, [nbr + 3])
                    pj1 = plsc.load_gather(tvm, [nbr + 4])
                    pj2 = plsc.load_gather(tvm, [nbr + 5])
                    rows = (lane + (cc * _K + q * 16)) * 8
                    plsc.store_scatter(rowbuf, [rows], xj0)
                    plsc.store_scatter(rowbuf, [rows + 1], xj1)
                    plsc.store_scatter(rowbuf, [rows + 2], xj2)
                    plsc.store_scatter(rowbuf, [rows + 3], pj0 - ccx)
                    plsc.store_scatter(rowbuf, [rows + 4], pj1 - ccy)
                    plsc.store_scatter(rowbuf, [rows + 5], pj2 - ccz)
            g0 = base + blk * RB
            pltpu.async_copy(rowbuf,
                             out_hbm.at[pl.ds(g0 * _K * 8, RB * _K * 8)],
                             sem_o).wait()

        z16 = jnp.zeros((16,), jnp.float32)
        for r in range(RB * _K // 16):
            rows0 = (lane + r * 16) * 8
            plsc.store_scatter(rowbuf, [rows0 + 6], z16)
            plsc.store_scatter(rowbuf, [rows0 + 7], z16)

        start(0, buf_a, sem_a)
        start(1, buf_b, sem_b)

        def loop_body(i, carry):
            wait(buf_a, sem_a)
            process(2 * i, buf_a)
            start(2 * i + 2, buf_a, sem_a)
            wait(buf_b, sem_b)
            process(2 * i + 1, buf_b)
            start(2 * i + 3, buf_b, sem_b)
            return carry

        jax.lax.fori_loop(0, nblocks // 2, loop_body, 0)
        wait(buf_a, sem_a)
        wait(buf_b, sem_b)

    kern = pl.kernel(
        body,
        out_type=pltpu.HBM((_K * N * 8,), jnp.float32),
        mesh=_SC_MESH,
        compiler_params=pltpu.CompilerParams(needs_layout_passes=False),
        scratch_types=[
            pltpu.VMEM((_P * 8,), jnp.float32),
            pltpu.VMEM((cpt * 8,), jnp.float32),
            pltpu.VMEM((RB, Pn), jnp.int32),
            pltpu.VMEM((RB, Pn), jnp.int32),
            pltpu.VMEM((RB * _K * 8,), jnp.float32),
            pltpu.SemaphoreType.DMA,
            pltpu.SemaphoreType.DMA,
            pltpu.SemaphoreType.DMA,
            pltpu.SemaphoreType.DMA,
        ],
    )
    return kern(rank2d, t1f, c3f)


def _sc_sa2(rank2d, xt, p1f, c3f):
    N, Pn = _B * _N2, _N1
    cpt = N // _NW
    RB = 4
    nblocks = cpt // RB
    n_per_cloud = _N2

    def body(rank_hbm, xt_hbm, p1_hbm, c3_hbm, outx_hbm, outr_hbm,
             pvm, cvm, buf_a, buf_b, nbrbuf, rowsbuf, relbuf,
             sem_a, sem_b, sem_t, sem_g, sem_o):
        wid = jax.lax.axis_index("s") * 2 + jax.lax.axis_index("c")
        base = wid * cpt
        cloud = base // n_per_cloud
        lane = jax.lax.iota(jnp.int32, 16)

        pltpu.async_copy(p1_hbm.at[pl.ds(cloud * Pn * 4, Pn * 4)], pvm,
                         sem_t).wait()
        pltpu.async_copy(c3_hbm.at[pl.ds(base * 8, cpt * 8)], cvm,
                         sem_t).wait()

        z16 = jnp.zeros((16,), jnp.float32)
        def zinit(r, carry):
            plsc.store_scatter(relbuf, [lane + r * 16], z16)
            return carry
        jax.lax.fori_loop(0, RB * _K, zinit, 0)

        def start(blk, buf, sem):
            blk = jnp.minimum(blk, nblocks - 1)
            pltpu.async_copy(rank_hbm.at[pl.ds(base + blk * RB, RB)], buf,
                             sem)

        def wait(buf, sem):
            pltpu.make_async_copy(rank_hbm.at[pl.ds(0, RB)], buf, sem).wait()

        def process(blk, buf):
            for cc in range(RB):
                g = base + blk * RB + cc
                local = g - base
                lastv = buf[cc, pl.ds(Pn - 16, 16)]
                nval = jnp.minimum(jnp.max(lastv), _K)
                row_splat = jnp.full((16,), cc, jnp.int32)
                lsplat = jnp.full((16,), local, jnp.int32)
                ccx = plsc.load_gather(cvm, [lsplat * 8])
                ccy = plsc.load_gather(cvm, [lsplat * 8 + 1])
                ccz = plsc.load_gather(cvm, [lsplat * 8 + 2])
                for q in range(4):
                    sv = lane + (16 * q)
                    c = _bsearch(buf, row_splat, sv, Pn)
                    nbr = jnp.where(sv < nval, c, 0)
                    pj0 = plsc.load_gather(pvm, [nbr * 4])
                    pj1 = plsc.load_gather(pvm, [nbr * 4 + 1])
                    pj2 = plsc.load_gather(pvm, [nbr * 4 + 2])
                    rows = (lane + (cc * _K + q * 16)) * 16
                    plsc.store_scatter(relbuf, [rows], pj0 - ccx)
                    plsc.store_scatter(relbuf, [rows + 1], pj1 - ccy)
                    plsc.store_scatter(relbuf, [rows + 2], pj2 - ccz)
                    pos = cc * _K + q * 16
                    nbrbuf[pos // 128, pl.ds(pos % 128, 16)] = nbr + cloud * Pn
            g0 = base + blk * RB
            for h in range(RB * _K // 128):
                pltpu.async_copy(xt_hbm.at[nbrbuf.at[h]], rowsbuf,
                                 sem_g).wait()
                pltpu.async_copy(rowsbuf,
                                 outx_hbm.at[pl.ds(g0 * _K + h * 128, 128)],
                                 sem_o).wait()
            pltpu.async_copy(relbuf,
                             outr_hbm.at[pl.ds(g0 * _K * 16, RB * _K * 16)],
                             sem_o).wait()

        start(0, buf_a, sem_a)
        start(1, buf_b, sem_b)

        def loop_body(i, carry):
            wait(buf_a, sem_a)
            process(2 * i, buf_a)
            start(2 * i + 2, buf_a, sem_a)
            wait(buf_b, sem_b)
            process(2 * i + 1, buf_b)
            start(2 * i + 3, buf_b, sem_b)
            return carry

        jax.lax.fori_loop(0, nblocks // 2, loop_body, 0)
        wait(buf_a, sem_a)
        wait(buf_b, sem_b)

    kern = pl.kernel(
        body,
        out_type=[
            pltpu.HBM((_K * N, 128), jnp.float32),
            pltpu.HBM((_K * N * 16,), jnp.float32),
        ],
        mesh=_SC_MESH,
        compiler_params=pltpu.CompilerParams(needs_layout_passes=False),
        scratch_types=[
            pltpu.VMEM((_N1 * 4,), jnp.float32),
            pltpu.VMEM((cpt * 8,), jnp.float32),
            pltpu.VMEM((RB, Pn), jnp.int32),
            pltpu.VMEM((RB, Pn), jnp.int32),
            pltpu.VMEM((RB * _K // 128, 128), jnp.int32),
            pltpu.VMEM((128, 128), jnp.float32),
            pltpu.VMEM((RB * _K * 16,), jnp.float32),
            pltpu.SemaphoreType.DMA,
            pltpu.SemaphoreType.DMA,
            pltpu.SemaphoreType.DMA,
            pltpu.SemaphoreType.DMA,
            pltpu.SemaphoreType.DMA,
        ],
    )
    return kern(rank2d, xt, p1f, c3f)


def _mlp_max_body(nlayers, Kn, bc, fin, f_ref, nv_ref, *refs):
    out_ref = refs[-1]
    prefs = refs[:-1]
    z = f_ref[...].reshape(bc * Kn, fin)
    for li in range(nlayers):
        wt = prefs[4 * li][...]
        b = prefs[4 * li + 1][...]
        sc = prefs[4 * li + 2][...]
        sh = prefs[4 * li + 3][...]
        z = jnp.dot(z, wt, preferred_element_type=jnp.float32) + b
        z = jnp.maximum(z, 0.0) * sc + sh
    fout = z.shape[1]
    z3 = z.reshape(bc, Kn, fout)
    nv = nv_ref[...]
    slot3 = jax.lax.broadcasted_iota(jnp.int32, (bc, Kn, 1), 1)
    nv3 = jnp.broadcast_to(nv[:, :, None], (bc, 1, 1))
    z3 = jnp.where(slot3 < nv3, z3, _NEG)
    k = Kn
    while k > 1:
        k //= 2
        z3 = jnp.maximum(z3[:, :k], z3[:, k:])
    z2 = z3.reshape(bc, fout)
    out_ref[...] = jnp.where(nv > 0, z2, 0.0)


def _mlp_max_call(feats, nvalid, players, bc):
    N, Kn, fin = feats.shape
    fout = players[-1][0].shape[1]
    nblk = N // bc
    body = functools.partial(_mlp_max_body, len(players), Kn, bc, fin)
    in_specs = [
        pl.BlockSpec((bc, Kn, fin), lambda i: (i, 0, 0)),
        pl.BlockSpec((bc, 1), lambda i: (i, 0)),
    ]
    args = [feats, nvalid]
    for p in players:
        for a in p:
            in_specs.append(pl.BlockSpec(a.shape, lambda i: (0, 0)))
            args.append(a)
    return pl.pallas_call(
        body,
        grid=(nblk,),
        in_specs=in_specs,
        out_specs=pl.BlockSpec((bc, fout), lambda i: (i, 0)),
        out_shape=jax.ShapeDtypeStruct((N, fout), jnp.float32),
    )(*args)


def _global_body(f_ref, *refs):
    out_ref = refs[-1]
    prefs = refs[:-1]
    z = f_ref[...]
    for li in range(3):
        wt = prefs[4 * li][...]
        b = prefs[4 * li + 1][...]
        sc = prefs[4 * li + 2][...]
        sh = prefs[4 * li + 3][...]
        z = jnp.dot(z, wt, preferred_element_type=jnp.float32) + b
        z = jnp.maximum(z, 0.0) * sc + sh
    rows = z.shape[0]
    while rows > _B:
        rows //= 2
        z = jnp.maximum(z[:rows], z[rows:])
    base = 12
    for li in range(3):
        wt = prefs[base + 2 * li][...]
        b = prefs[base + 2 * li + 1][...]
        z = jnp.dot(z, wt, preferred_element_type=jnp.float32) + b
        if li < 2:
            z = jnp.maximum(z, 0.0)
    out_ref[...] = z


def _global_call(feat_t, players3, lins):
    args = [feat_t]
    for p in players3:
        args.extend(p)
    for (W, b) in lins:
        args.append(W.T)
        args.append(b[None, :])
    return pl.pallas_call(
        _global_body,
        out_shape=jax.ShapeDtypeStruct((_B, 32), jnp.float32),
    )(*args)


def _pad_first_layer(players, fin, dpad):
    w1t, b1, sc1, sh1 = players[0]
    f1 = w1t.shape[1]
    wpad = jnp.zeros((dpad, f1), jnp.float32)
    wpad = wpad.at[:fin].set(w1t)
    return [(wpad, b1, sc1, sh1)] + list(players[1:])


def kernel(x, pos, batch, params):
    p1l = _prep_layers(params['mlp1'])
    p2l = _prep_layers(params['mlp2'])
    p3l = _prep_layers(params['mlp3'])

    xb = x.reshape(_B, _P, 3)
    pb = pos.reshape(_B, _P, 3)
    px = pb[:, :, 0]
    py = pb[:, :, 1]
    pz = pb[:, :, 2]
    cx1, cy1, cz1, cx2, cy2, cz2 = _fps_call(px, py, pz)

    rank1, nv1 = _rank_call(cx1.T, cy1.T, cz1.T, px, py, pz, _R1, 128)
    t1 = jnp.concatenate([xb, pb, jnp.zeros((_B, _P, 2), jnp.float32)],
                         axis=-1).reshape(_B * _P, 8)
    c1s = jnp.stack([cx1, cy1, cz1], axis=-1).reshape(_B * _N1, 3)
    c13 = jnp.concatenate([c1s, jnp.zeros((_B * _N1, 5), jnp.float32)],
                          axis=1)
    feats1 = _sc_sa1(rank1.reshape(_B * _N1, _P), t1.reshape(-1),
                     c13.reshape(-1))
    feats1 = feats1.reshape(_B * _N1, _K, 8)
    x1 = _mlp_max_call(feats1, nv1.reshape(_B * _N1, 1),
                       _pad_first_layer(p1l, 6, 8), 128)

    rank2, nv2 = _rank_call(cx2.T, cy2.T, cz2.T, cx1, cy1, cz1, _R2, 128)
    p1c = jnp.stack([cx1, cy1, cz1,
                     jnp.zeros((_B, _N1), jnp.float32)], axis=-1)
    c2s = jnp.stack([cx2, cy2, cz2], axis=-1).reshape(_B * _N2, 3)
    c23 = jnp.concatenate([c2s, jnp.zeros((_B * _N2, 5), jnp.float32)],
                          axis=1)
    fx2, fr2 = _sc_sa2(rank2.reshape(_B * _N2, _N1), x1, p1c.reshape(-1),
                       c23.reshape(-1))
    fr2 = fr2.reshape(_K * _B * _N2, 16)
    feats2 = jnp.concatenate([fx2, fr2], axis=-1)
    feats2 = feats2.reshape(_B * _N2, _K, 144)
    pl2 = _pad_first_layer(p2l, 131, 144)
    x2 = _mlp_max_call(feats2, nv2.reshape(_B * _N2, 1), pl2, 128)

    feat3 = jnp.concatenate(
        [x2.reshape(_B, _N2, 256), jnp.stack([cx2, cy2, cz2], axis=-1)],
        axis=-1)
    feat3_t = feat3.transpose(1, 0, 2).reshape(_N2 * _B, 259)
    lins = [params['lin1'], params['lin2'], params['lin3']]
    return _global_call(feat3_t, p3l, lins)
